# Initial kernel scaffold; baseline (speedup 1.0000x reference)
import jax
import jax.numpy as jnp
from jax import lax
from jax.experimental import pallas as pl
from jax.experimental.pallas import tpu as pltpu
from jax.experimental.pallas import tpu_sc as plsc

N = 10000
D = 128
E = 320000
NCLS = 40
EPS = 1e-5
NC, NS = 2, 16
NW = NC * NS
EPW = E // NW
B = 80
NB = EPW // B
RPT = N // NS

BLK = 2000
GRID = N // BLK


def _mesh():
    return plsc.VectorSubcoreMesh(
        core_axis_name="c", subcore_axis_name="s", num_cores=NC, num_subcores=NS)


def _deg_body(dst_hbm, zero_hbm, ones_hbm, out_hbm, acc, dst_v, ones_v):
    c = lax.axis_index("c")
    s = lax.axis_index("s")
    wid = c * NS + s
    pltpu.sync_copy(zero_hbm.at[pl.ds(s * RPT, RPT)], acc.at[pl.ds(s * RPT, RPT)])
    pltpu.sync_copy(dst_hbm.at[wid], dst_v)
    pltpu.sync_copy(ones_hbm, ones_v)
    plsc.subcore_barrier()

    def body(j, carry):
        pltpu.sync_copy(ones_v, acc.at[dst_v.at[j]], add=True)
        return carry

    lax.fori_loop(0, NB, body, 0)
    plsc.subcore_barrier()
    pltpu.sync_copy(acc.at[pl.ds(s * RPT, RPT)],
                    out_hbm.at[c].at[pl.ds(s * RPT, RPT)])


def _sc_deg(dst3, zeros16, ones16):
    fn = pl.kernel(
        _deg_body,
        out_type=jax.ShapeDtypeStruct((NC, N, 16), jnp.float32),
        mesh=_mesh(),
        scratch_types=[
            pltpu.VMEM_SHARED((N, 16), jnp.float32),
            pltpu.VMEM((NB, B), jnp.int32),
            pltpu.VMEM((B, 16), jnp.float32),
        ],
    )
    return fn(dst3, zeros16, ones16)


def _spmm_body(xw_hbm, src_hbm, dst_hbm, zero_hbm, out_hbm,
               acc, src_v, dst_v, rows, gsem0, gsem1):
    c = lax.axis_index("c")
    s = lax.axis_index("s")
    wid = c * NS + s
    pltpu.sync_copy(zero_hbm.at[pl.ds(s * RPT, RPT)], acc.at[pl.ds(s * RPT, RPT)])
    pltpu.sync_copy(src_hbm.at[wid], src_v)
    pltpu.sync_copy(dst_hbm.at[wid], dst_v)
    plsc.subcore_barrier()

    pltpu.async_copy(xw_hbm.at[src_v.at[0]], rows.at[0], gsem0)
    pltpu.async_copy(xw_hbm.at[src_v.at[1]], rows.at[1], gsem1)

    def pair(j0, carry):
        j = j0 * 2
        pltpu.make_async_copy(xw_hbm.at[src_v.at[j]], rows.at[0], gsem0).wait()
        pltpu.sync_copy(rows.at[0], acc.at[dst_v.at[j]], add=True)
        pltpu.async_copy(xw_hbm.at[src_v.at[j + 2]], rows.at[0], gsem0)
        pltpu.make_async_copy(xw_hbm.at[src_v.at[j + 1]], rows.at[1], gsem1).wait()
        pltpu.sync_copy(rows.at[1], acc.at[dst_v.at[j + 1]], add=True)

        @pl.when(j + 3 < NB)
        def _():
            pltpu.async_copy(xw_hbm.at[src_v.at[j + 3]], rows.at[1], gsem1)

        return carry

    lax.fori_loop(0, (NB - 1) // 2, pair, 0)
    pltpu.make_async_copy(xw_hbm.at[src_v.at[NB - 1]], rows.at[0], gsem0).wait()
    pltpu.sync_copy(rows.at[0], acc.at[dst_v.at[NB - 1]], add=True)
    plsc.subcore_barrier()
    pltpu.sync_copy(acc.at[pl.ds(s * RPT, RPT)],
                    out_hbm.at[c].at[pl.ds(s * RPT, RPT)])


def _sc_spmm(xw, src3, dst3, zeros_nd):
    fn = pl.kernel(
        _spmm_body,
        out_type=jax.ShapeDtypeStruct((NC, N, D), jnp.float32),
        mesh=_mesh(),
        scratch_types=[
            pltpu.VMEM_SHARED((N, D), jnp.float32),
            pltpu.VMEM((NB, B), jnp.int32),
            pltpu.VMEM((NB, B), jnp.int32),
            pltpu.VMEM((2, B, D), jnp.float32),
            pltpu.SemaphoreType.DMA,
            pltpu.SemaphoreType.DMA,
        ],
    )
    return fn(xw, src3, dst3, zeros_nd)


def _pre_body(x_ref, degp_ref, sum_ref, sq_ref, dinv_ref, acc_s, acc_q):
    i = pl.program_id(0)
    blk = x_ref[...]
    s = jnp.sum(blk, axis=0, keepdims=True)
    q = jnp.sum(blk * blk, axis=0, keepdims=True)

    @pl.when(i == 0)
    def _():
        acc_s[...] = s
        acc_q[...] = q

    @pl.when(i > 0)
    def _():
        acc_s[...] += s
        acc_q[...] += q

    @pl.when(i == GRID - 1)
    def _():
        sum_ref[...] = acc_s[...]
        sq_ref[...] = acc_q[...]

    cnt = degp_ref[0][:, 0:1] + degp_ref[1][:, 0:1]
    dinv_ref[...] = lax.rsqrt(cnt + 1.0)


def _k_pre(x, degp):
    return pl.pallas_call(
        _pre_body,
        grid=(GRID,),
        in_specs=[
            pl.BlockSpec((BLK, D), lambda i: (i, 0)),
            pl.BlockSpec((NC, BLK, 16), lambda i: (0, i, 0)),
        ],
        out_specs=[
            pl.BlockSpec((1, D), lambda i: (0, 0)),
            pl.BlockSpec((1, D), lambda i: (0, 0)),
            pl.BlockSpec((BLK, 1), lambda i: (i, 0)),
        ],
        out_shape=[
            jax.ShapeDtypeStruct((1, D), jnp.float32),
            jax.ShapeDtypeStruct((1, D), jnp.float32),
            jax.ShapeDtypeStruct((N, 1), jnp.float32),
        ],
        scratch_shapes=[pltpu.VMEM((1, D), jnp.float32),
                        pltpu.VMEM((1, D), jnp.float32)],
    )(x, degp)


def _mm_body(y_ref, sum_ref, sq_ref, g_ref, b_ref, dinv_ref, w_ref, out_ref):
    m = sum_ref[...] / N
    v = sq_ref[...] / N - m * m
    h = (y_ref[...] - m) * (lax.rsqrt(v + EPS) * g_ref[...]) + b_ref[...]
    h = h * dinv_ref[...]
    out_ref[...] = jnp.dot(h, w_ref[...], preferred_element_type=jnp.float32)


def _k_mm(y, ysum, ysq, g2, b2, dinv, W):
    return pl.pallas_call(
        _mm_body,
        grid=(GRID,),
        in_specs=[
            pl.BlockSpec((BLK, D), lambda i: (i, 0)),
            pl.BlockSpec((1, D), lambda i: (0, 0)),
            pl.BlockSpec((1, D), lambda i: (0, 0)),
            pl.BlockSpec((1, D), lambda i: (0, 0)),
            pl.BlockSpec((1, D), lambda i: (0, 0)),
            pl.BlockSpec((BLK, 1), lambda i: (i, 0)),
            pl.BlockSpec((D, D), lambda i: (0, 0)),
        ],
        out_specs=pl.BlockSpec((BLK, D), lambda i: (i, 0)),
        out_shape=jax.ShapeDtypeStruct((N, D), jnp.float32),
    )(y, ysum, ysq, g2, b2, dinv, W)


def _comb_body(sp_ref, xw_ref, dinv_ref, b_ref, y_ref, sum_ref, sq_ref,
               acc_s, acc_q):
    i = pl.program_id(0)
    y = sp_ref[0] + sp_ref[1] + xw_ref[...]
    y = jnp.maximum(y * dinv_ref[...] + b_ref[...], 0.0)
    y_ref[...] = y
    s = jnp.sum(y, axis=0, keepdims=True)
    q = jnp.sum(y * y, axis=0, keepdims=True)

    @pl.when(i == 0)
    def _():
        acc_s[...] = s
        acc_q[...] = q

    @pl.when(i > 0)
    def _():
        acc_s[...] += s
        acc_q[...] += q

    @pl.when(i == GRID - 1)
    def _():
        sum_ref[...] = acc_s[...]
        sq_ref[...] = acc_q[...]


def _k_comb(sp, xw, dinv, b2):
    return pl.pallas_call(
        _comb_body,
        grid=(GRID,),
        in_specs=[
            pl.BlockSpec((NC, BLK, D), lambda i: (0, i, 0)),
            pl.BlockSpec((BLK, D), lambda i: (i, 0)),
            pl.BlockSpec((BLK, 1), lambda i: (i, 0)),
            pl.BlockSpec((1, D), lambda i: (0, 0)),
        ],
        out_specs=[
            pl.BlockSpec((BLK, D), lambda i: (i, 0)),
            pl.BlockSpec((1, D), lambda i: (0, 0)),
            pl.BlockSpec((1, D), lambda i: (0, 0)),
        ],
        out_shape=[
            jax.ShapeDtypeStruct((N, D), jnp.float32),
            jax.ShapeDtypeStruct((1, D), jnp.float32),
            jax.ShapeDtypeStruct((1, D), jnp.float32),
        ],
        scratch_shapes=[pltpu.VMEM((1, D), jnp.float32),
                        pltpu.VMEM((1, D), jnp.float32)],
    )(sp, xw, dinv, b2)


def _final_body(sp_ref, xw_ref, dinv_ref, b_ref, w_ref, ob_ref, out_ref):
    y = sp_ref[0] + sp_ref[1] + xw_ref[...]
    y = jnp.maximum(y * dinv_ref[...] + b_ref[...], 0.0)
    out_ref[...] = jnp.dot(y, w_ref[...], preferred_element_type=jnp.float32) + ob_ref[...]


def _k_final(sp, xw, dinv, b2, wp, obp):
    return pl.pallas_call(
        _final_body,
        grid=(GRID,),
        in_specs=[
            pl.BlockSpec((NC, BLK, D), lambda i: (0, i, 0)),
            pl.BlockSpec((BLK, D), lambda i: (i, 0)),
            pl.BlockSpec((BLK, 1), lambda i: (i, 0)),
            pl.BlockSpec((1, D), lambda i: (0, 0)),
            pl.BlockSpec((D, D), lambda i: (0, 0)),
            pl.BlockSpec((1, D), lambda i: (0, 0)),
        ],
        out_specs=pl.BlockSpec((BLK, D), lambda i: (i, 0)),
        out_shape=jax.ShapeDtypeStruct((N, D), jnp.float32),
    )(sp, xw, dinv, b2, wp, obp)


def kernel(x, edge_index, bn1_g, bn1_b, W1, b1, bn2_g, bn2_b, W2, b2,
           bn3_g, bn3_b, W3, b3, out_W, out_b):
    f32 = jnp.float32
    src3 = edge_index[0].reshape(NW, NB, B)
    dst3 = edge_index[1].reshape(NW, NB, B)
    zeros_nd = jnp.zeros((N, D), f32)
    zeros16 = jnp.zeros((N, 16), f32)
    ones16 = jnp.ones((B, 16), f32)
    wp = jnp.zeros((D, D), f32).at[:, :NCLS].set(out_W)
    obp = jnp.zeros((1, D), f32).at[0, :NCLS].set(out_b)
    r2 = lambda a: a.reshape(1, D)

    degp = _sc_deg(dst3, zeros16, ones16)
    xsum, xsq, dinv = _k_pre(x, degp)

    xw1 = _k_mm(x, xsum, xsq, r2(bn1_g), r2(bn1_b), dinv, W1)
    sp1 = _sc_spmm(xw1, src3, dst3, zeros_nd)
    y1, s1, q1 = _k_comb(sp1, xw1, dinv, r2(b1))

    xw2 = _k_mm(y1, s1, q1, r2(bn2_g), r2(bn2_b), dinv, W2)
    sp2 = _sc_spmm(xw2, src3, dst3, zeros_nd)
    y2, s2, q2 = _k_comb(sp2, xw2, dinv, r2(b2))

    xw3 = _k_mm(y2, s2, q2, r2(bn3_g), r2(bn3_b), dinv, W3)
    sp3 = _sc_spmm(xw3, src3, dst3, zeros_nd)
    logits = _k_final(sp3, xw3, dinv, r2(b3), wp, obp)
    return logits[:, :NCLS]


# speedup vs baseline: 21.3431x; 21.3431x over previous
import jax
import jax.numpy as jnp
from jax import lax
from jax.experimental import pallas as pl
from jax.experimental.pallas import tpu as pltpu
from jax.experimental.pallas import tpu_sc as plsc

N = 10000
D = 128
E = 320000
NCLS = 40
EPS = 1e-5
NC, NS = 2, 16
NW = NC * NS
EPW = E // NW
B = 80
NB = EPW // B
NP = 10240
RPT = NP // NS

BLK = 2000
GRID = N // BLK


def _mesh():
    return plsc.VectorSubcoreMesh(
        core_axis_name="c", subcore_axis_name="s", num_cores=NC, num_subcores=NS)


def _deg_body(dst_hbm, zero_hbm, ones_hbm, out_hbm, acc, dst_v, ones_v):
    c = lax.axis_index("c")
    s = lax.axis_index("s")
    wid = c * NS + s
    pltpu.sync_copy(zero_hbm.at[pl.ds(s * RPT, RPT)], acc.at[pl.ds(s * RPT, RPT)])
    pltpu.sync_copy(dst_hbm.at[wid], dst_v)
    pltpu.sync_copy(ones_hbm, ones_v)
    plsc.subcore_barrier()

    def body(j, carry):
        pltpu.sync_copy(ones_v, acc.at[dst_v.at[j]], add=True)
        return carry

    lax.fori_loop(0, NB, body, 0)
    plsc.subcore_barrier()
    pltpu.sync_copy(acc.at[pl.ds(s * RPT, RPT)],
                    out_hbm.at[c].at[pl.ds(s * RPT, RPT)])


def _sc_deg(dst3, zeros16, ones16):
    fn = pl.kernel(
        _deg_body,
        out_type=jax.ShapeDtypeStruct((NC, NP, D), jnp.float32),
        mesh=_mesh(),
        scratch_types=[
            pltpu.VMEM_SHARED((NP, D), jnp.float32),
            pltpu.VMEM((NB, B), jnp.int32),
            pltpu.VMEM((B, D), jnp.float32),
        ],
    )
    return fn(dst3, zeros16, ones16)


def _spmm_body(xw_hbm, src_hbm, dst_hbm, zero_hbm, out_hbm,
               acc, src_v, dst_v, rows, gsem0, gsem1):
    c = lax.axis_index("c")
    s = lax.axis_index("s")
    wid = c * NS + s
    pltpu.sync_copy(zero_hbm.at[pl.ds(s * RPT, RPT)], acc.at[pl.ds(s * RPT, RPT)])
    pltpu.sync_copy(src_hbm.at[wid], src_v)
    pltpu.sync_copy(dst_hbm.at[wid], dst_v)
    plsc.subcore_barrier()

    def sidx(j):
        return src_v.at[pl.ds(pl.multiple_of(j * B, 8), B)]

    pltpu.async_copy(xw_hbm.at[sidx(0)], rows.at[0], gsem0)
    pltpu.async_copy(xw_hbm.at[sidx(1)], rows.at[1], gsem1)

    def pair(j0, carry):
        j = j0 * 2
        pltpu.make_async_copy(xw_hbm.at[sidx(j)], rows.at[0], gsem0).wait()
        pltpu.sync_copy(rows.at[0], acc.at[dst_v.at[j]], add=True)
        pltpu.async_copy(xw_hbm.at[sidx(j + 2)], rows.at[0], gsem0)
        pltpu.make_async_copy(xw_hbm.at[sidx(j + 1)], rows.at[1], gsem1).wait()
        pltpu.sync_copy(rows.at[1], acc.at[dst_v.at[j + 1]], add=True)

        @pl.when(j + 3 < NB)
        def _():
            pltpu.async_copy(xw_hbm.at[sidx(j + 3)], rows.at[1], gsem1)

        return carry

    lax.fori_loop(0, (NB - 1) // 2, pair, 0)
    pltpu.make_async_copy(xw_hbm.at[sidx(NB - 1)], rows.at[0], gsem0).wait()
    pltpu.sync_copy(rows.at[0], acc.at[dst_v.at[NB - 1]], add=True)
    plsc.subcore_barrier()
    pltpu.sync_copy(acc.at[pl.ds(s * RPT, RPT)],
                    out_hbm.at[c].at[pl.ds(s * RPT, RPT)])


def _sc_spmm(xw, src3, dst3, zeros_nd):
    fn = pl.kernel(
        _spmm_body,
        out_type=jax.ShapeDtypeStruct((NC, NP, D), jnp.float32),
        mesh=_mesh(),
        scratch_types=[
            pltpu.VMEM_SHARED((NP, D), jnp.float32),
            pltpu.VMEM((EPW,), jnp.int32),
            pltpu.VMEM((NB, B), jnp.int32),
            pltpu.VMEM((2, B, D), jnp.float32),
            pltpu.SemaphoreType.DMA,
            pltpu.SemaphoreType.DMA,
        ],
    )
    return fn(xw, src3, dst3, zeros_nd)


def _pre_body(x_ref, degp_ref, sum_ref, sq_ref, dinv_ref, acc_s, acc_q):
    i = pl.program_id(0)
    blk = x_ref[...]
    s = jnp.sum(blk, axis=0, keepdims=True)
    q = jnp.sum(blk * blk, axis=0, keepdims=True)

    @pl.when(i == 0)
    def _():
        acc_s[...] = s
        acc_q[...] = q

    @pl.when(i > 0)
    def _():
        acc_s[...] += s
        acc_q[...] += q

    @pl.when(i == GRID - 1)
    def _():
        sum_ref[...] = acc_s[...]
        sq_ref[...] = acc_q[...]

    cnt = degp_ref[0][:, 0:1] + degp_ref[1][:, 0:1]
    dinv_ref[...] = lax.rsqrt(cnt + 1.0)


def _k_pre(x, degp):
    return pl.pallas_call(
        _pre_body,
        grid=(GRID,),
        in_specs=[
            pl.BlockSpec((BLK, D), lambda i: (i, 0)),
            pl.BlockSpec((NC, BLK, D), lambda i: (0, i, 0)),
        ],
        out_specs=[
            pl.BlockSpec((1, D), lambda i: (0, 0)),
            pl.BlockSpec((1, D), lambda i: (0, 0)),
            pl.BlockSpec((BLK, 1), lambda i: (i, 0)),
        ],
        out_shape=[
            jax.ShapeDtypeStruct((1, D), jnp.float32),
            jax.ShapeDtypeStruct((1, D), jnp.float32),
            jax.ShapeDtypeStruct((N, 1), jnp.float32),
        ],
        scratch_shapes=[pltpu.VMEM((1, D), jnp.float32),
                        pltpu.VMEM((1, D), jnp.float32)],
    )(x, degp)


def _mm_body(y_ref, sum_ref, sq_ref, g_ref, b_ref, dinv_ref, w_ref, out_ref):
    m = sum_ref[...] / N
    v = sq_ref[...] / N - m * m
    h = (y_ref[...] - m) * (lax.rsqrt(v + EPS) * g_ref[...]) + b_ref[...]
    h = h * dinv_ref[...]
    out_ref[...] = jnp.dot(h, w_ref[...], preferred_element_type=jnp.float32)


def _k_mm(y, ysum, ysq, g2, b2, dinv, W):
    return pl.pallas_call(
        _mm_body,
        grid=(GRID,),
        in_specs=[
            pl.BlockSpec((BLK, D), lambda i: (i, 0)),
            pl.BlockSpec((1, D), lambda i: (0, 0)),
            pl.BlockSpec((1, D), lambda i: (0, 0)),
            pl.BlockSpec((1, D), lambda i: (0, 0)),
            pl.BlockSpec((1, D), lambda i: (0, 0)),
            pl.BlockSpec((BLK, 1), lambda i: (i, 0)),
            pl.BlockSpec((D, D), lambda i: (0, 0)),
        ],
        out_specs=pl.BlockSpec((BLK, D), lambda i: (i, 0)),
        out_shape=jax.ShapeDtypeStruct((N, D), jnp.float32),
    )(y, ysum, ysq, g2, b2, dinv, W)


def _comb_body(sp_ref, xw_ref, dinv_ref, b_ref, y_ref, sum_ref, sq_ref,
               acc_s, acc_q):
    i = pl.program_id(0)
    y = sp_ref[0] + sp_ref[1] + xw_ref[...]
    y = jnp.maximum(y * dinv_ref[...] + b_ref[...], 0.0)
    y_ref[...] = y
    s = jnp.sum(y, axis=0, keepdims=True)
    q = jnp.sum(y * y, axis=0, keepdims=True)

    @pl.when(i == 0)
    def _():
        acc_s[...] = s
        acc_q[...] = q

    @pl.when(i > 0)
    def _():
        acc_s[...] += s
        acc_q[...] += q

    @pl.when(i == GRID - 1)
    def _():
        sum_ref[...] = acc_s[...]
        sq_ref[...] = acc_q[...]


def _k_comb(sp, xw, dinv, b2):
    return pl.pallas_call(
        _comb_body,
        grid=(GRID,),
        in_specs=[
            pl.BlockSpec((NC, BLK, D), lambda i: (0, i, 0)),
            pl.BlockSpec((BLK, D), lambda i: (i, 0)),
            pl.BlockSpec((BLK, 1), lambda i: (i, 0)),
            pl.BlockSpec((1, D), lambda i: (0, 0)),
        ],
        out_specs=[
            pl.BlockSpec((BLK, D), lambda i: (i, 0)),
            pl.BlockSpec((1, D), lambda i: (0, 0)),
            pl.BlockSpec((1, D), lambda i: (0, 0)),
        ],
        out_shape=[
            jax.ShapeDtypeStruct((N, D), jnp.float32),
            jax.ShapeDtypeStruct((1, D), jnp.float32),
            jax.ShapeDtypeStruct((1, D), jnp.float32),
        ],
        scratch_shapes=[pltpu.VMEM((1, D), jnp.float32),
                        pltpu.VMEM((1, D), jnp.float32)],
    )(sp, xw, dinv, b2)


def _final_body(sp_ref, xw_ref, dinv_ref, b_ref, w_ref, ob_ref, out_ref):
    y = sp_ref[0] + sp_ref[1] + xw_ref[...]
    y = jnp.maximum(y * dinv_ref[...] + b_ref[...], 0.0)
    out_ref[...] = jnp.dot(y, w_ref[...], preferred_element_type=jnp.float32) + ob_ref[...]


def _k_final(sp, xw, dinv, b2, wp, obp):
    return pl.pallas_call(
        _final_body,
        grid=(GRID,),
        in_specs=[
            pl.BlockSpec((NC, BLK, D), lambda i: (0, i, 0)),
            pl.BlockSpec((BLK, D), lambda i: (i, 0)),
            pl.BlockSpec((BLK, 1), lambda i: (i, 0)),
            pl.BlockSpec((1, D), lambda i: (0, 0)),
            pl.BlockSpec((D, D), lambda i: (0, 0)),
            pl.BlockSpec((1, D), lambda i: (0, 0)),
        ],
        out_specs=pl.BlockSpec((BLK, D), lambda i: (i, 0)),
        out_shape=jax.ShapeDtypeStruct((N, D), jnp.float32),
    )(sp, xw, dinv, b2, wp, obp)


def kernel(x, edge_index, bn1_g, bn1_b, W1, b1, bn2_g, bn2_b, W2, b2,
           bn3_g, bn3_b, W3, b3, out_W, out_b):
    f32 = jnp.float32
    src2 = edge_index[0].reshape(NW, EPW)
    dst3 = edge_index[1].reshape(NW, NB, B)
    zeros_nd = jnp.zeros((NP, D), f32)
    ones_bd = jnp.ones((B, D), f32)
    wp = jnp.zeros((D, D), f32).at[:, :NCLS].set(out_W)
    obp = jnp.zeros((1, D), f32).at[0, :NCLS].set(out_b)
    r2 = lambda a: a.reshape(1, D)

    degp = _sc_deg(dst3, zeros_nd, ones_bd)
    xsum, xsq, dinv = _k_pre(x, degp)

    xw1 = _k_mm(x, xsum, xsq, r2(bn1_g), r2(bn1_b), dinv, W1)
    sp1 = _sc_spmm(xw1, src2, dst3, zeros_nd)
    y1, s1, q1 = _k_comb(sp1, xw1, dinv, r2(b1))

    xw2 = _k_mm(y1, s1, q1, r2(bn2_g), r2(bn2_b), dinv, W2)
    sp2 = _sc_spmm(xw2, src2, dst3, zeros_nd)
    y2, s2, q2 = _k_comb(sp2, xw2, dinv, r2(b2))

    xw3 = _k_mm(y2, s2, q2, r2(bn3_g), r2(bn3_b), dinv, W3)
    sp3 = _sc_spmm(xw3, src2, dst3, zeros_nd)
    logits = _k_final(sp3, xw3, dinv, r2(b3), wp, obp)
    return logits[:, :NCLS]


# speedup vs baseline: 21.3744x; 1.0015x over previous
import jax
import jax.numpy as jnp
from jax import lax
from jax.experimental import pallas as pl
from jax.experimental.pallas import tpu as pltpu
from jax.experimental.pallas import tpu_sc as plsc

N = 10000
D = 128
E = 320000
NCLS = 40
EPS = 1e-5
NC, NS = 2, 16
NW = NC * NS
EPW = E // NW
B = 80
NB = EPW // B
NP = 10240
RPT = NP // NS

BLK = 2000
GRID = N // BLK


def _mesh():
    return plsc.VectorSubcoreMesh(
        core_axis_name="c", subcore_axis_name="s", num_cores=NC, num_subcores=NS)


def _deg_body(dst_hbm, zero_hbm, ones_hbm, out_hbm, acc, dst_v, ones_v):
    c = lax.axis_index("c")
    s = lax.axis_index("s")
    wid = c * NS + s
    pltpu.sync_copy(zero_hbm.at[pl.ds(s * RPT, RPT)], acc.at[pl.ds(s * RPT, RPT)])
    pltpu.sync_copy(dst_hbm.at[wid], dst_v)
    pltpu.sync_copy(ones_hbm, ones_v)
    plsc.subcore_barrier()

    def body(j, carry):
        pltpu.sync_copy(ones_v, acc.at[dst_v.at[j]], add=True)
        return carry

    lax.fori_loop(0, NB, body, 0)
    plsc.subcore_barrier()
    pltpu.sync_copy(acc.at[pl.ds(s * RPT, RPT)],
                    out_hbm.at[c].at[pl.ds(s * RPT, RPT)])


def _sc_deg(dst3, zeros_nd, ones_bd):
    fn = pl.kernel(
        _deg_body,
        out_type=jax.ShapeDtypeStruct((NC, NP, D), jnp.float32),
        mesh=_mesh(),
        scratch_types=[
            pltpu.VMEM_SHARED((NP, D), jnp.float32),
            pltpu.VMEM((NB, B), jnp.int32),
            pltpu.VMEM((B, D), jnp.float32),
        ],
    )
    return fn(dst3, zeros_nd, ones_bd)


def _spmm_body(xw_hbm, src_hbm, dst_hbm, zero_hbm, out_hbm,
               acc, src_v, dst_v, rows, gsem0, gsem1):
    c = lax.axis_index("c")
    s = lax.axis_index("s")
    wid = c * NS + s
    pltpu.sync_copy(zero_hbm.at[pl.ds(s * RPT, RPT)], acc.at[pl.ds(s * RPT, RPT)])
    pltpu.sync_copy(src_hbm.at[wid], src_v)
    pltpu.sync_copy(dst_hbm.at[wid], dst_v)
    plsc.subcore_barrier()

    def sidx(j):
        return src_v.at[pl.ds(pl.multiple_of(j * B, 8), B)]

    pltpu.async_copy(xw_hbm.at[sidx(0)], rows.at[0], gsem0)
    pltpu.async_copy(xw_hbm.at[sidx(1)], rows.at[1], gsem1)

    def pair(j0, carry):
        j = j0 * 2
        pltpu.make_async_copy(xw_hbm.at[sidx(j)], rows.at[0], gsem0).wait()
        pltpu.sync_copy(rows.at[0], acc.at[dst_v.at[j]], add=True)
        pltpu.async_copy(xw_hbm.at[sidx(j + 2)], rows.at[0], gsem0)
        pltpu.make_async_copy(xw_hbm.at[sidx(j + 1)], rows.at[1], gsem1).wait()
        pltpu.sync_copy(rows.at[1], acc.at[dst_v.at[j + 1]], add=True)

        @pl.when(j + 3 < NB)
        def _():
            pltpu.async_copy(xw_hbm.at[sidx(j + 3)], rows.at[1], gsem1)

        return carry

    lax.fori_loop(0, (NB - 1) // 2, pair, 0)
    pltpu.make_async_copy(xw_hbm.at[sidx(NB - 1)], rows.at[0], gsem0).wait()
    pltpu.sync_copy(rows.at[0], acc.at[dst_v.at[NB - 1]], add=True)
    plsc.subcore_barrier()
    pltpu.sync_copy(acc.at[pl.ds(s * RPT, RPT)],
                    out_hbm.at[c].at[pl.ds(s * RPT, RPT)])


def _sc_spmm(xw, src3, dst3, zeros_nd):
    fn = pl.kernel(
        _spmm_body,
        out_type=jax.ShapeDtypeStruct((NC, NP, D), jnp.float32),
        mesh=_mesh(),
        scratch_types=[
            pltpu.VMEM_SHARED((NP, D), jnp.float32),
            pltpu.VMEM((EPW,), jnp.int32),
            pltpu.VMEM((NB, B), jnp.int32),
            pltpu.VMEM((2, B, D), jnp.float32),
            pltpu.SemaphoreType.DMA,
            pltpu.SemaphoreType.DMA,
        ],
    )
    return fn(xw, src3, dst3, zeros_nd)


def _prep_body(x_ref, degp_ref, g_ref, b_ref, w_ref, xw_ref, dinv_ref,
               x_sc, acc_s, acc_q):
    p = pl.program_id(0)
    i = pl.program_id(1)
    cnt = degp_ref[0][:, 0:1] + degp_ref[1][:, 0:1]
    dv = lax.rsqrt(cnt + 1.0)
    dinv_ref[...] = dv

    @pl.when(p == 0)
    def _():
        blk = x_ref[...]
        x_sc[pl.ds(i * BLK, BLK), :] = blk
        s = jnp.sum(blk, axis=0, keepdims=True)
        q = jnp.sum(blk * blk, axis=0, keepdims=True)

        @pl.when(i == 0)
        def _():
            acc_s[...] = s
            acc_q[...] = q

        @pl.when(i > 0)
        def _():
            acc_s[...] += s
            acc_q[...] += q

    @pl.when(p == 1)
    def _():
        m = acc_s[...] / N
        v = acc_q[...] / N - m * m
        h = (x_sc[pl.ds(i * BLK, BLK), :] - m) * (lax.rsqrt(v + EPS) * g_ref[...]) + b_ref[...]
        h = h * dv
        xw_ref[...] = jnp.dot(h, w_ref[...], preferred_element_type=jnp.float32,
                              precision=lax.Precision.HIGHEST)


def _k_prep(x, degp, g2, b2, W):
    return pl.pallas_call(
        _prep_body,
        grid=(2, GRID),
        in_specs=[
            pl.BlockSpec((BLK, D), lambda p, i: (i * (1 - p), 0)),
            pl.BlockSpec((NC, BLK, D), lambda p, i: (0, i, 0)),
            pl.BlockSpec((1, D), lambda p, i: (0, 0)),
            pl.BlockSpec((1, D), lambda p, i: (0, 0)),
            pl.BlockSpec((D, D), lambda p, i: (0, 0)),
        ],
        out_specs=[
            pl.BlockSpec((BLK, D), lambda p, i: (i * p, 0)),
            pl.BlockSpec((BLK, 1), lambda p, i: (i, 0)),
        ],
        out_shape=[
            jax.ShapeDtypeStruct((N, D), jnp.float32),
            jax.ShapeDtypeStruct((N, 1), jnp.float32),
        ],
        scratch_shapes=[pltpu.VMEM((N, D), jnp.float32),
                        pltpu.VMEM((1, D), jnp.float32),
                        pltpu.VMEM((1, D), jnp.float32)],
    )(x, degp, g2, b2, W)


def _layer_body(sp_ref, xw_ref, dinv_ref, bc_ref, g_ref, b_ref, w_ref,
                out_ref, y_sc, acc_s, acc_q):
    p = pl.program_id(0)
    i = pl.program_id(1)

    @pl.when(p == 0)
    def _():
        y = sp_ref[0] + sp_ref[1] + xw_ref[...]
        y = jnp.maximum(y * dinv_ref[...] + bc_ref[...], 0.0)
        y_sc[pl.ds(i * BLK, BLK), :] = y
        s = jnp.sum(y, axis=0, keepdims=True)
        q = jnp.sum(y * y, axis=0, keepdims=True)

        @pl.when(i == 0)
        def _():
            acc_s[...] = s
            acc_q[...] = q

        @pl.when(i > 0)
        def _():
            acc_s[...] += s
            acc_q[...] += q

    @pl.when(p == 1)
    def _():
        m = acc_s[...] / N
        v = acc_q[...] / N - m * m
        h = (y_sc[pl.ds(i * BLK, BLK), :] - m) * (lax.rsqrt(v + EPS) * g_ref[...]) + b_ref[...]
        h = h * dinv_ref[...]
        out_ref[...] = jnp.dot(h, w_ref[...], preferred_element_type=jnp.float32,
                               precision=lax.Precision.HIGHEST)


def _k_layer(sp, xw, dinv, bc2, g2, b2, W):
    return pl.pallas_call(
        _layer_body,
        grid=(2, GRID),
        in_specs=[
            pl.BlockSpec((NC, BLK, D), lambda p, i: (0, i * (1 - p), 0)),
            pl.BlockSpec((BLK, D), lambda p, i: (i * (1 - p), 0)),
            pl.BlockSpec((BLK, 1), lambda p, i: (i, 0)),
            pl.BlockSpec((1, D), lambda p, i: (0, 0)),
            pl.BlockSpec((1, D), lambda p, i: (0, 0)),
            pl.BlockSpec((1, D), lambda p, i: (0, 0)),
            pl.BlockSpec((D, D), lambda p, i: (0, 0)),
        ],
        out_specs=pl.BlockSpec((BLK, D), lambda p, i: (i * p, 0)),
        out_shape=jax.ShapeDtypeStruct((N, D), jnp.float32),
        scratch_shapes=[pltpu.VMEM((N, D), jnp.float32),
                        pltpu.VMEM((1, D), jnp.float32),
                        pltpu.VMEM((1, D), jnp.float32)],
    )(sp, xw, dinv, bc2, g2, b2, W)


def _final_body(sp_ref, xw_ref, dinv_ref, b_ref, w_ref, ob_ref, out_ref):
    y = sp_ref[0] + sp_ref[1] + xw_ref[...]
    y = jnp.maximum(y * dinv_ref[...] + b_ref[...], 0.0)
    out_ref[...] = jnp.dot(y, w_ref[...], preferred_element_type=jnp.float32,
                           precision=lax.Precision.HIGHEST) + ob_ref[...]


def _k_final(sp, xw, dinv, b2, wp, obp):
    return pl.pallas_call(
        _final_body,
        grid=(GRID,),
        in_specs=[
            pl.BlockSpec((NC, BLK, D), lambda i: (0, i, 0)),
            pl.BlockSpec((BLK, D), lambda i: (i, 0)),
            pl.BlockSpec((BLK, 1), lambda i: (i, 0)),
            pl.BlockSpec((1, D), lambda i: (0, 0)),
            pl.BlockSpec((D, D), lambda i: (0, 0)),
            pl.BlockSpec((1, D), lambda i: (0, 0)),
        ],
        out_specs=pl.BlockSpec((BLK, D), lambda i: (i, 0)),
        out_shape=jax.ShapeDtypeStruct((N, D), jnp.float32),
    )(sp, xw, dinv, b2, wp, obp)


def kernel(x, edge_index, bn1_g, bn1_b, W1, b1, bn2_g, bn2_b, W2, b2,
           bn3_g, bn3_b, W3, b3, out_W, out_b):
    f32 = jnp.float32
    src2 = edge_index[0].reshape(NW, EPW)
    dst3 = edge_index[1].reshape(NW, NB, B)
    zeros_nd = jnp.zeros((NP, D), f32)
    ones_bd = jnp.ones((B, D), f32)
    wp = jnp.zeros((D, D), f32).at[:, :NCLS].set(out_W)
    obp = jnp.zeros((1, D), f32).at[0, :NCLS].set(out_b)
    r2 = lambda a: a.reshape(1, D)

    degp = _sc_deg(dst3, zeros_nd, ones_bd)
    xw1, dinv = _k_prep(x, degp, r2(bn1_g), r2(bn1_b), W1)
    sp1 = _sc_spmm(xw1, src2, dst3, zeros_nd)
    xw2 = _k_layer(sp1, xw1, dinv, r2(b1), r2(bn2_g), r2(bn2_b), W2)
    sp2 = _sc_spmm(xw2, src2, dst3, zeros_nd)
    xw3 = _k_layer(sp2, xw2, dinv, r2(b2), r2(bn3_g), r2(bn3_b), W3)
    sp3 = _sc_spmm(xw3, src2, dst3, zeros_nd)
    logits = _k_final(sp3, xw3, dinv, r2(b3), wp, obp)
    return logits[:, :NCLS]


# speedup vs baseline: 21.5025x; 1.0060x over previous
import jax
import jax.numpy as jnp
from jax import lax
from jax.experimental import pallas as pl
from jax.experimental.pallas import tpu as pltpu
from jax.experimental.pallas import tpu_sc as plsc

N = 10000
D = 128
E = 320000
NCLS = 40
EPS = 1e-5
NC, NS = 2, 16
NW = NC * NS
EPW = E // NW
B = 80
NB = EPW // B
NP = 10240
RPT = NP // NS

BLK = 2000
GRID = N // BLK


def _mesh():
    return plsc.VectorSubcoreMesh(
        core_axis_name="c", subcore_axis_name="s", num_cores=NC, num_subcores=NS)


def _deg_body(dst_hbm, zero_hbm, ones_hbm, out_hbm, acc, dst_v, ones_v):
    c = lax.axis_index("c")
    s = lax.axis_index("s")
    wid = c * NS + s
    pltpu.sync_copy(zero_hbm.at[pl.ds(s * RPT, RPT)], acc.at[pl.ds(s * RPT, RPT)])
    pltpu.sync_copy(dst_hbm.at[wid], dst_v)
    pltpu.sync_copy(ones_hbm, ones_v)
    plsc.subcore_barrier()

    def body(j, carry):
        pltpu.sync_copy(ones_v, acc.at[dst_v.at[j]], add=True)
        return carry

    lax.fori_loop(0, NB, body, 0)
    plsc.subcore_barrier()
    pltpu.sync_copy(acc.at[pl.ds(s * RPT, RPT)],
                    out_hbm.at[c].at[pl.ds(s * RPT, RPT)])


def _sc_deg(dst3, zeros_nd, ones_bd):
    fn = pl.kernel(
        _deg_body,
        out_type=jax.ShapeDtypeStruct((NC, NP, D), jnp.float32),
        mesh=_mesh(),
        scratch_types=[
            pltpu.VMEM_SHARED((NP, D), jnp.float32),
            pltpu.VMEM((NB, B), jnp.int32),
            pltpu.VMEM((B, D), jnp.float32),
        ],
    )
    return fn(dst3, zeros_nd, ones_bd)


def _spmm_body(xw_hbm, src_hbm, dst_hbm, zero_hbm, out_hbm,
               acc, src_v, dst_v, rows, gsem0, gsem1):
    c = lax.axis_index("c")
    s = lax.axis_index("s")
    wid = c * NS + s
    pltpu.sync_copy(zero_hbm.at[pl.ds(s * RPT, RPT)], acc.at[pl.ds(s * RPT, RPT)])
    pltpu.sync_copy(src_hbm.at[wid], src_v)
    pltpu.sync_copy(dst_hbm.at[wid], dst_v)
    plsc.subcore_barrier()

    def sidx(j):
        return src_v.at[pl.ds(pl.multiple_of(j * B, 8), B)]

    pltpu.async_copy(xw_hbm.at[sidx(0)], rows.at[0], gsem0)
    pltpu.async_copy(xw_hbm.at[sidx(1)], rows.at[1], gsem1)

    def pair(j0, carry):
        j = j0 * 2
        pltpu.make_async_copy(xw_hbm.at[sidx(j)], rows.at[0], gsem0).wait()
        pltpu.sync_copy(rows.at[0], acc.at[dst_v.at[j]], add=True)
        pltpu.async_copy(xw_hbm.at[sidx(j + 2)], rows.at[0], gsem0)
        pltpu.make_async_copy(xw_hbm.at[sidx(j + 1)], rows.at[1], gsem1).wait()
        pltpu.sync_copy(rows.at[1], acc.at[dst_v.at[j + 1]], add=True)

        @pl.when(j + 3 < NB)
        def _():
            pltpu.async_copy(xw_hbm.at[sidx(j + 3)], rows.at[1], gsem1)

        return carry

    lax.fori_loop(0, (NB - 1) // 2, pair, 0)
    pltpu.make_async_copy(xw_hbm.at[sidx(NB - 1)], rows.at[0], gsem0).wait()
    pltpu.sync_copy(rows.at[0], acc.at[dst_v.at[NB - 1]], add=True)
    plsc.subcore_barrier()
    pltpu.sync_copy(acc.at[pl.ds(s * RPT, RPT)],
                    out_hbm.at[c].at[pl.ds(s * RPT, RPT)])


def _sc_spmm(xw, src3, dst3, zeros_nd):
    fn = pl.kernel(
        _spmm_body,
        out_type=jax.ShapeDtypeStruct((NC, NP, D), jnp.float32),
        mesh=_mesh(),
        scratch_types=[
            pltpu.VMEM_SHARED((NP, D), jnp.float32),
            pltpu.VMEM((EPW,), jnp.int32),
            pltpu.VMEM((NB, B), jnp.int32),
            pltpu.VMEM((2, B, D), jnp.float32),
            pltpu.SemaphoreType.DMA,
            pltpu.SemaphoreType.DMA,
        ],
    )
    return fn(xw, src3, dst3, zeros_nd)


def _prep_body(x_ref, degp_ref, g_ref, b_ref, w_ref, xw_ref, dinv_ref,
               x_sc, dv_sc, acc_s, acc_q):
    p = pl.program_id(0)
    i = pl.program_id(1)

    @pl.when(p == 0)
    def _():
        cnt = degp_ref[0][:, 0:1] + degp_ref[1][:, 0:1]
        dv = lax.rsqrt(cnt + 1.0)
        dinv_ref[...] = dv
        dv_sc[pl.ds(i * BLK, BLK), :] = dv
        blk = x_ref[...]
        x_sc[pl.ds(i * BLK, BLK), :] = blk
        s = jnp.sum(blk, axis=0, keepdims=True)
        q = jnp.sum(blk * blk, axis=0, keepdims=True)

        @pl.when(i == 0)
        def _():
            acc_s[...] = s
            acc_q[...] = q

        @pl.when(i > 0)
        def _():
            acc_s[...] += s
            acc_q[...] += q

    @pl.when(p == 1)
    def _():
        m = acc_s[...] / N
        v = acc_q[...] / N - m * m
        h = (x_sc[pl.ds(i * BLK, BLK), :] - m) * (lax.rsqrt(v + EPS) * g_ref[...]) + b_ref[...]
        h = h * dv_sc[pl.ds(i * BLK, BLK), :]
        xw_ref[...] = jnp.dot(h, w_ref[...], preferred_element_type=jnp.float32,
                              precision=lax.Precision.HIGHEST)


def _k_prep(x, degp, g2, b2, W):
    return pl.pallas_call(
        _prep_body,
        grid=(2, GRID),
        in_specs=[
            pl.BlockSpec((BLK, D), lambda p, i: (i * (1 - p), 0)),
            pl.BlockSpec((NC, BLK, D), lambda p, i: (0, i * (1 - p), 0)),
            pl.BlockSpec((1, D), lambda p, i: (0, 0)),
            pl.BlockSpec((1, D), lambda p, i: (0, 0)),
            pl.BlockSpec((D, D), lambda p, i: (0, 0)),
        ],
        out_specs=[
            pl.BlockSpec((BLK, D), lambda p, i: (i * p, 0)),
            pl.BlockSpec((BLK, 1), lambda p, i: (i * (1 - p) + (GRID - 1) * p, 0)),
        ],
        out_shape=[
            jax.ShapeDtypeStruct((N, D), jnp.float32),
            jax.ShapeDtypeStruct((N, 1), jnp.float32),
        ],
        scratch_shapes=[pltpu.VMEM((N, D), jnp.float32),
                        pltpu.VMEM((N, 1), jnp.float32),
                        pltpu.VMEM((1, D), jnp.float32),
                        pltpu.VMEM((1, D), jnp.float32)],
    )(x, degp, g2, b2, W)


def _layer_body(sp_ref, xw_ref, dinv_ref, bc_ref, g_ref, b_ref, w_ref,
                out_ref, y_sc, acc_s, acc_q):
    p = pl.program_id(0)
    i = pl.program_id(1)

    @pl.when(p == 0)
    def _():
        y = sp_ref[0] + sp_ref[1] + xw_ref[...]
        y = jnp.maximum(y * dinv_ref[...] + bc_ref[...], 0.0)
        y_sc[pl.ds(i * BLK, BLK), :] = y
        s = jnp.sum(y, axis=0, keepdims=True)
        q = jnp.sum(y * y, axis=0, keepdims=True)

        @pl.when(i == 0)
        def _():
            acc_s[...] = s
            acc_q[...] = q

        @pl.when(i > 0)
        def _():
            acc_s[...] += s
            acc_q[...] += q

    @pl.when(p == 1)
    def _():
        m = acc_s[...] / N
        v = acc_q[...] / N - m * m
        h = (y_sc[pl.ds(i * BLK, BLK), :] - m) * (lax.rsqrt(v + EPS) * g_ref[...]) + b_ref[...]
        h = h * dinv_ref[...]
        out_ref[...] = jnp.dot(h, w_ref[...], preferred_element_type=jnp.float32,
                               precision=lax.Precision.HIGHEST)


def _k_layer(sp, xw, dinv, bc2, g2, b2, W):
    return pl.pallas_call(
        _layer_body,
        grid=(2, GRID),
        in_specs=[
            pl.BlockSpec((NC, BLK, D), lambda p, i: (0, i * (1 - p), 0)),
            pl.BlockSpec((BLK, D), lambda p, i: (i * (1 - p), 0)),
            pl.BlockSpec((BLK, 1), lambda p, i: (i, 0)),
            pl.BlockSpec((1, D), lambda p, i: (0, 0)),
            pl.BlockSpec((1, D), lambda p, i: (0, 0)),
            pl.BlockSpec((1, D), lambda p, i: (0, 0)),
            pl.BlockSpec((D, D), lambda p, i: (0, 0)),
        ],
        out_specs=pl.BlockSpec((BLK, D), lambda p, i: (i * p, 0)),
        out_shape=jax.ShapeDtypeStruct((N, D), jnp.float32),
        scratch_shapes=[pltpu.VMEM((N, D), jnp.float32),
                        pltpu.VMEM((1, D), jnp.float32),
                        pltpu.VMEM((1, D), jnp.float32)],
    )(sp, xw, dinv, bc2, g2, b2, W)


def _final_body(sp_ref, xw_ref, dinv_ref, b_ref, w_ref, ob_ref, out_ref):
    y = sp_ref[0] + sp_ref[1] + xw_ref[...]
    y = jnp.maximum(y * dinv_ref[...] + b_ref[...], 0.0)
    out_ref[...] = jnp.dot(y, w_ref[...], preferred_element_type=jnp.float32,
                           precision=lax.Precision.HIGHEST) + ob_ref[...]


def _k_final(sp, xw, dinv, b2, wp, obp):
    return pl.pallas_call(
        _final_body,
        grid=(GRID,),
        in_specs=[
            pl.BlockSpec((NC, BLK, D), lambda i: (0, i, 0)),
            pl.BlockSpec((BLK, D), lambda i: (i, 0)),
            pl.BlockSpec((BLK, 1), lambda i: (i, 0)),
            pl.BlockSpec((1, D), lambda i: (0, 0)),
            pl.BlockSpec((D, NCLS), lambda i: (0, 0)),
            pl.BlockSpec((1, NCLS), lambda i: (0, 0)),
        ],
        out_specs=pl.BlockSpec((BLK, NCLS), lambda i: (i, 0)),
        out_shape=jax.ShapeDtypeStruct((N, NCLS), jnp.float32),
    )(sp, xw, dinv, b2, wp, obp)


def kernel(x, edge_index, bn1_g, bn1_b, W1, b1, bn2_g, bn2_b, W2, b2,
           bn3_g, bn3_b, W3, b3, out_W, out_b):
    f32 = jnp.float32
    src2 = edge_index[0].reshape(NW, EPW)
    dst3 = edge_index[1].reshape(NW, NB, B)
    zeros_nd = jnp.zeros((NP, D), f32)
    ones_bd = jnp.ones((B, D), f32)
    wp = out_W
    obp = out_b.reshape(1, NCLS)
    r2 = lambda a: a.reshape(1, D)

    degp = _sc_deg(dst3, zeros_nd, ones_bd)
    xw1, dinv = _k_prep(x, degp, r2(bn1_g), r2(bn1_b), W1)
    sp1 = _sc_spmm(xw1, src2, dst3, zeros_nd)
    xw2 = _k_layer(sp1, xw1, dinv, r2(b1), r2(bn2_g), r2(bn2_b), W2)
    sp2 = _sc_spmm(xw2, src2, dst3, zeros_nd)
    xw3 = _k_layer(sp2, xw2, dinv, r2(b2), r2(bn3_g), r2(bn3_b), W3)
    sp3 = _sc_spmm(xw3, src2, dst3, zeros_nd)
    return _k_final(sp3, xw3, dinv, r2(b3), wp, obp)


# speedup vs baseline: 21.7864x; 1.0132x over previous
import jax
import jax.numpy as jnp
from jax import lax
from jax.experimental import pallas as pl
from jax.experimental.pallas import tpu as pltpu
from jax.experimental.pallas import tpu_sc as plsc

N = 10000
D = 128
E = 320000
NCLS = 40
EPS = 1e-5
NC, NS = 2, 16
NW = NC * NS
EPW = E // NW
B = 80
NB = EPW // B
NP = 10240
RPT = NP // NS

BLK = 2000
GRID = N // BLK


def _mesh():
    return plsc.VectorSubcoreMesh(
        core_axis_name="c", subcore_axis_name="s", num_cores=NC, num_subcores=NS)


def _deg_body(dst_hbm, zero_hbm, ones_hbm, out_hbm, acc, dst_v, ones_v,
              sem0, sem1, sem2):
    c = lax.axis_index("c")
    s = lax.axis_index("s")
    wid = c * NS + s
    cz = pltpu.async_copy(zero_hbm.at[pl.ds(s * RPT, RPT)],
                          acc.at[pl.ds(s * RPT, RPT)], sem0)
    cd = pltpu.async_copy(dst_hbm.at[wid], dst_v, sem1)
    co = pltpu.async_copy(ones_hbm, ones_v, sem2)
    cz.wait()
    cd.wait()
    co.wait()
    plsc.subcore_barrier()

    def body(j, carry):
        pltpu.sync_copy(ones_v, acc.at[dst_v.at[j]], add=True)
        return carry

    lax.fori_loop(0, NB, body, 0)
    plsc.subcore_barrier()
    pltpu.sync_copy(acc.at[pl.ds(s * RPT, RPT)],
                    out_hbm.at[c].at[pl.ds(s * RPT, RPT)])


def _sc_deg(dst3, zeros_nd, ones_bd):
    fn = pl.kernel(
        _deg_body,
        out_type=jax.ShapeDtypeStruct((NC, NP, D), jnp.float32),
        mesh=_mesh(),
        scratch_types=[
            pltpu.VMEM_SHARED((NP, D), jnp.float32),
            pltpu.VMEM((NB, B), jnp.int32),
            pltpu.VMEM((B, D), jnp.float32),
            pltpu.SemaphoreType.DMA,
            pltpu.SemaphoreType.DMA,
            pltpu.SemaphoreType.DMA,
        ],
    )
    return fn(dst3, zeros_nd, ones_bd)


def _spmm_body(xw_hbm, src_hbm, dst_hbm, zero_hbm, out_hbm,
               acc, src_v, dst_v, rows, gsem0, gsem1, ssem):
    c = lax.axis_index("c")
    s = lax.axis_index("s")
    wid = c * NS + s
    cz = pltpu.async_copy(zero_hbm.at[pl.ds(s * RPT, RPT)],
                          acc.at[pl.ds(s * RPT, RPT)], gsem0)
    cs = pltpu.async_copy(src_hbm.at[wid], src_v, gsem1)
    cd = pltpu.async_copy(dst_hbm.at[wid], dst_v, ssem)
    cz.wait()
    cs.wait()
    cd.wait()
    plsc.subcore_barrier()

    def sidx(j):
        return src_v.at[pl.ds(pl.multiple_of(j * B, 8), B)]

    pltpu.async_copy(xw_hbm.at[sidx(0)], rows.at[0], gsem0)
    pltpu.async_copy(xw_hbm.at[sidx(1)], rows.at[1], gsem1)

    def pair(j0, carry):
        j = j0 * 2
        pltpu.make_async_copy(xw_hbm.at[sidx(j)], rows.at[0], gsem0).wait()
        pltpu.sync_copy(rows.at[0], acc.at[dst_v.at[j]], add=True)
        pltpu.async_copy(xw_hbm.at[sidx(j + 2)], rows.at[0], gsem0)
        pltpu.make_async_copy(xw_hbm.at[sidx(j + 1)], rows.at[1], gsem1).wait()
        pltpu.sync_copy(rows.at[1], acc.at[dst_v.at[j + 1]], add=True)

        @pl.when(j + 3 < NB)
        def _():
            pltpu.async_copy(xw_hbm.at[sidx(j + 3)], rows.at[1], gsem1)

        return carry

    lax.fori_loop(0, (NB - 1) // 2, pair, 0)
    pltpu.make_async_copy(xw_hbm.at[sidx(NB - 1)], rows.at[0], gsem0).wait()
    pltpu.sync_copy(rows.at[0], acc.at[dst_v.at[NB - 1]], add=True)
    plsc.subcore_barrier()
    pltpu.sync_copy(acc.at[pl.ds(s * RPT, RPT)],
                    out_hbm.at[c].at[pl.ds(s * RPT, RPT)])


def _sc_spmm(xw, src3, dst3, zeros_nd):
    fn = pl.kernel(
        _spmm_body,
        out_type=jax.ShapeDtypeStruct((NC, NP, D), jnp.float32),
        mesh=_mesh(),
        scratch_types=[
            pltpu.VMEM_SHARED((NP, D), jnp.float32),
            pltpu.VMEM((EPW,), jnp.int32),
            pltpu.VMEM((NB, B), jnp.int32),
            pltpu.VMEM((2, B, D), jnp.float32),
            pltpu.SemaphoreType.DMA,
            pltpu.SemaphoreType.DMA,
            pltpu.SemaphoreType.DMA,
        ],
    )
    return fn(xw, src3, dst3, zeros_nd)


def _prep_body(x_ref, degp_ref, g_ref, b_ref, w_ref, xw_ref, dinv_ref,
               x_sc, dv_sc, acc_s, acc_q):
    p = pl.program_id(0)
    i = pl.program_id(1)

    @pl.when(p == 0)
    def _():
        cnt = degp_ref[0][:, 0:1] + degp_ref[1][:, 0:1]
        dv = lax.rsqrt(cnt + 1.0)
        dinv_ref[...] = dv
        dv_sc[pl.ds(i * BLK, BLK), :] = dv
        blk = x_ref[...]
        x_sc[pl.ds(i * BLK, BLK), :] = blk
        s = jnp.sum(blk, axis=0, keepdims=True)
        q = jnp.sum(blk * blk, axis=0, keepdims=True)

        @pl.when(i == 0)
        def _():
            acc_s[...] = s
            acc_q[...] = q

        @pl.when(i > 0)
        def _():
            acc_s[...] += s
            acc_q[...] += q

    @pl.when(p == 1)
    def _():
        m = acc_s[...] / N
        v = acc_q[...] / N - m * m
        h = (x_sc[pl.ds(i * BLK, BLK), :] - m) * (lax.rsqrt(v + EPS) * g_ref[...]) + b_ref[...]
        h = h * dv_sc[pl.ds(i * BLK, BLK), :]
        xw_ref[...] = jnp.dot(h, w_ref[...], preferred_element_type=jnp.float32,
                              precision=lax.Precision.HIGHEST)


def _k_prep(x, degp, g2, b2, W):
    return pl.pallas_call(
        _prep_body,
        grid=(2, GRID),
        in_specs=[
            pl.BlockSpec((BLK, D), lambda p, i: (i * (1 - p), 0)),
            pl.BlockSpec((NC, BLK, D), lambda p, i: (0, i * (1 - p), 0)),
            pl.BlockSpec((1, D), lambda p, i: (0, 0)),
            pl.BlockSpec((1, D), lambda p, i: (0, 0)),
            pl.BlockSpec((D, D), lambda p, i: (0, 0)),
        ],
        out_specs=[
            pl.BlockSpec((BLK, D), lambda p, i: (i * p, 0)),
            pl.BlockSpec((BLK, 1), lambda p, i: (i * (1 - p) + (GRID - 1) * p, 0)),
        ],
        out_shape=[
            jax.ShapeDtypeStruct((N, D), jnp.float32),
            jax.ShapeDtypeStruct((N, 1), jnp.float32),
        ],
        scratch_shapes=[pltpu.VMEM((N, D), jnp.float32),
                        pltpu.VMEM((N, 1), jnp.float32),
                        pltpu.VMEM((1, D), jnp.float32),
                        pltpu.VMEM((1, D), jnp.float32)],
    )(x, degp, g2, b2, W)


def _layer_body(sp_ref, xw_ref, dinv_ref, bc_ref, g_ref, b_ref, w_ref,
                out_ref, y_sc, acc_s, acc_q):
    p = pl.program_id(0)
    i = pl.program_id(1)

    @pl.when(p == 0)
    def _():
        y = sp_ref[0] + sp_ref[1] + xw_ref[...]
        y = jnp.maximum(y * dinv_ref[...] + bc_ref[...], 0.0)
        y_sc[pl.ds(i * BLK, BLK), :] = y
        s = jnp.sum(y, axis=0, keepdims=True)
        q = jnp.sum(y * y, axis=0, keepdims=True)

        @pl.when(i == 0)
        def _():
            acc_s[...] = s
            acc_q[...] = q

        @pl.when(i > 0)
        def _():
            acc_s[...] += s
            acc_q[...] += q

    @pl.when(p == 1)
    def _():
        m = acc_s[...] / N
        v = acc_q[...] / N - m * m
        h = (y_sc[pl.ds(i * BLK, BLK), :] - m) * (lax.rsqrt(v + EPS) * g_ref[...]) + b_ref[...]
        h = h * dinv_ref[...]
        out_ref[...] = jnp.dot(h, w_ref[...], preferred_element_type=jnp.float32,
                               precision=lax.Precision.HIGHEST)


def _k_layer(sp, xw, dinv, bc2, g2, b2, W):
    return pl.pallas_call(
        _layer_body,
        grid=(2, GRID),
        in_specs=[
            pl.BlockSpec((NC, BLK, D), lambda p, i: (0, i * (1 - p), 0)),
            pl.BlockSpec((BLK, D), lambda p, i: (i * (1 - p), 0)),
            pl.BlockSpec((BLK, 1), lambda p, i: (i, 0)),
            pl.BlockSpec((1, D), lambda p, i: (0, 0)),
            pl.BlockSpec((1, D), lambda p, i: (0, 0)),
            pl.BlockSpec((1, D), lambda p, i: (0, 0)),
            pl.BlockSpec((D, D), lambda p, i: (0, 0)),
        ],
        out_specs=pl.BlockSpec((BLK, D), lambda p, i: (i * p, 0)),
        out_shape=jax.ShapeDtypeStruct((N, D), jnp.float32),
        scratch_shapes=[pltpu.VMEM((N, D), jnp.float32),
                        pltpu.VMEM((1, D), jnp.float32),
                        pltpu.VMEM((1, D), jnp.float32)],
    )(sp, xw, dinv, bc2, g2, b2, W)


def _final_body(sp_ref, xw_ref, dinv_ref, b_ref, w_ref, ob_ref, out_ref):
    y = sp_ref[0] + sp_ref[1] + xw_ref[...]
    y = jnp.maximum(y * dinv_ref[...] + b_ref[...], 0.0)
    out_ref[...] = jnp.dot(y, w_ref[...], preferred_element_type=jnp.float32,
                           precision=lax.Precision.HIGHEST) + ob_ref[...]


def _k_final(sp, xw, dinv, b2, wp, obp):
    return pl.pallas_call(
        _final_body,
        grid=(GRID,),
        in_specs=[
            pl.BlockSpec((NC, BLK, D), lambda i: (0, i, 0)),
            pl.BlockSpec((BLK, D), lambda i: (i, 0)),
            pl.BlockSpec((BLK, 1), lambda i: (i, 0)),
            pl.BlockSpec((1, D), lambda i: (0, 0)),
            pl.BlockSpec((D, NCLS), lambda i: (0, 0)),
            pl.BlockSpec((1, NCLS), lambda i: (0, 0)),
        ],
        out_specs=pl.BlockSpec((BLK, NCLS), lambda i: (i, 0)),
        out_shape=jax.ShapeDtypeStruct((N, NCLS), jnp.float32),
    )(sp, xw, dinv, b2, wp, obp)


def kernel(x, edge_index, bn1_g, bn1_b, W1, b1, bn2_g, bn2_b, W2, b2,
           bn3_g, bn3_b, W3, b3, out_W, out_b):
    f32 = jnp.float32
    src2 = edge_index[0].reshape(NW, EPW)
    dst3 = edge_index[1].reshape(NW, NB, B)
    zeros_nd = jnp.zeros((NP, D), f32)
    ones_bd = jnp.ones((B, D), f32)
    wp = out_W
    obp = out_b.reshape(1, NCLS)
    r2 = lambda a: a.reshape(1, D)

    degp = _sc_deg(dst3, zeros_nd, ones_bd)
    xw1, dinv = _k_prep(x, degp, r2(bn1_g), r2(bn1_b), W1)
    sp1 = _sc_spmm(xw1, src2, dst3, zeros_nd)
    xw2 = _k_layer(sp1, xw1, dinv, r2(b1), r2(bn2_g), r2(bn2_b), W2)
    sp2 = _sc_spmm(xw2, src2, dst3, zeros_nd)
    xw3 = _k_layer(sp2, xw2, dinv, r2(b2), r2(bn3_g), r2(bn3_b), W3)
    sp3 = _sc_spmm(xw3, src2, dst3, zeros_nd)
    return _k_final(sp3, xw3, dinv, r2(b3), wp, obp)


# speedup vs baseline: 21.8705x; 1.0039x over previous
import jax
import jax.numpy as jnp
from jax import lax
from jax.experimental import pallas as pl
from jax.experimental.pallas import tpu as pltpu
from jax.experimental.pallas import tpu_sc as plsc

N = 10000
D = 128
E = 320000
NCLS = 40
EPS = 1e-5
NC, NS = 2, 16
NW = NC * NS
EPW = E // NW
B = 80
NB = EPW // B
NP = 10240
RPT = NP // NS

BLK = 2000
GRID = N // BLK


def _mesh():
    return plsc.VectorSubcoreMesh(
        core_axis_name="c", subcore_axis_name="s", num_cores=NC, num_subcores=NS)


def _deg_body(dst_hbm, zero_hbm, ones_hbm, out_hbm, acc, dst_v, ones_v,
              sem0, sem1, sem2):
    c = lax.axis_index("c")
    s = lax.axis_index("s")
    wid = c * NS + s
    cz = pltpu.async_copy(zero_hbm.at[pl.ds(s * RPT, RPT)],
                          acc.at[pl.ds(s * RPT, RPT)], sem0)
    cd = pltpu.async_copy(dst_hbm.at[wid], dst_v, sem1)
    co = pltpu.async_copy(ones_hbm, ones_v, sem2)
    cz.wait()
    cd.wait()
    co.wait()
    plsc.subcore_barrier()

    pltpu.async_copy(ones_v, acc.at[dst_v.at[0]], sem0, add=True)
    pltpu.async_copy(ones_v, acc.at[dst_v.at[1]], sem1, add=True)

    def body(j0, carry):
        j = j0 * 2
        pltpu.make_async_copy(ones_v, acc.at[dst_v.at[j]], sem0).wait()

        @pl.when(j + 2 < NB)
        def _():
            pltpu.async_copy(ones_v, acc.at[dst_v.at[j + 2]], sem0, add=True)

        @pl.when(j + 1 < NB)
        def _():
            pltpu.make_async_copy(ones_v, acc.at[dst_v.at[j + 1]], sem1).wait()

            @pl.when(j + 3 < NB)
            def _():
                pltpu.async_copy(ones_v, acc.at[dst_v.at[j + 3]], sem1, add=True)

        return carry

    lax.fori_loop(0, (NB + 1) // 2, body, 0)
    plsc.subcore_barrier()
    pltpu.sync_copy(acc.at[pl.ds(s * RPT, RPT)],
                    out_hbm.at[c].at[pl.ds(s * RPT, RPT)])


def _sc_deg(dst3, zeros_nd, ones_bd):
    fn = pl.kernel(
        _deg_body,
        out_type=jax.ShapeDtypeStruct((NC, NP, D), jnp.float32),
        mesh=_mesh(),
        scratch_types=[
            pltpu.VMEM_SHARED((NP, D), jnp.float32),
            pltpu.VMEM((NB, B), jnp.int32),
            pltpu.VMEM((B, D), jnp.float32),
            pltpu.SemaphoreType.DMA,
            pltpu.SemaphoreType.DMA,
            pltpu.SemaphoreType.DMA,
        ],
    )
    return fn(dst3, zeros_nd, ones_bd)


def _spmm_body(xw_hbm, src_hbm, dst_hbm, zero_hbm, out_hbm,
               acc, src_v, dst_v, rows, gsem0, gsem1, ssem):
    c = lax.axis_index("c")
    s = lax.axis_index("s")
    wid = c * NS + s
    cz = pltpu.async_copy(zero_hbm.at[pl.ds(s * RPT, RPT)],
                          acc.at[pl.ds(s * RPT, RPT)], gsem0)
    cs = pltpu.async_copy(src_hbm.at[wid], src_v, gsem1)
    cd = pltpu.async_copy(dst_hbm.at[wid], dst_v, ssem)
    cz.wait()
    cs.wait()
    cd.wait()
    plsc.subcore_barrier()

    def sidx(j):
        return src_v.at[pl.ds(pl.multiple_of(j * B, 8), B)]

    pltpu.async_copy(xw_hbm.at[sidx(0)], rows.at[0], gsem0)
    pltpu.async_copy(xw_hbm.at[sidx(1)], rows.at[1], gsem1)

    def pair(j0, carry):
        j = j0 * 2
        pltpu.make_async_copy(xw_hbm.at[sidx(j)], rows.at[0], gsem0).wait()
        pltpu.sync_copy(rows.at[0], acc.at[dst_v.at[j]], add=True)
        pltpu.async_copy(xw_hbm.at[sidx(j + 2)], rows.at[0], gsem0)
        pltpu.make_async_copy(xw_hbm.at[sidx(j + 1)], rows.at[1], gsem1).wait()
        pltpu.sync_copy(rows.at[1], acc.at[dst_v.at[j + 1]], add=True)

        @pl.when(j + 3 < NB)
        def _():
            pltpu.async_copy(xw_hbm.at[sidx(j + 3)], rows.at[1], gsem1)

        return carry

    lax.fori_loop(0, (NB - 1) // 2, pair, 0)
    pltpu.make_async_copy(xw_hbm.at[sidx(NB - 1)], rows.at[0], gsem0).wait()
    pltpu.sync_copy(rows.at[0], acc.at[dst_v.at[NB - 1]], add=True)
    plsc.subcore_barrier()
    pltpu.sync_copy(acc.at[pl.ds(s * RPT, RPT)],
                    out_hbm.at[c].at[pl.ds(s * RPT, RPT)])


def _sc_spmm(xw, src3, dst3, zeros_nd):
    fn = pl.kernel(
        _spmm_body,
        out_type=jax.ShapeDtypeStruct((NC, NP, D), jnp.float32),
        mesh=_mesh(),
        scratch_types=[
            pltpu.VMEM_SHARED((NP, D), jnp.float32),
            pltpu.VMEM((EPW,), jnp.int32),
            pltpu.VMEM((NB, B), jnp.int32),
            pltpu.VMEM((2, B, D), jnp.float32),
            pltpu.SemaphoreType.DMA,
            pltpu.SemaphoreType.DMA,
            pltpu.SemaphoreType.DMA,
        ],
    )
    return fn(xw, src3, dst3, zeros_nd)


def _prep_body(x_ref, degp_ref, g_ref, b_ref, w_ref, xw_ref, dinv_ref,
               x_sc, dv_sc, acc_s, acc_q):
    p = pl.program_id(0)
    i = pl.program_id(1)

    @pl.when(p == 0)
    def _():
        cnt = degp_ref[0][:, 0:1] + degp_ref[1][:, 0:1]
        dv = lax.rsqrt(cnt + 1.0)
        dinv_ref[...] = dv
        dv_sc[pl.ds(i * BLK, BLK), :] = dv
        blk = x_ref[...]
        x_sc[pl.ds(i * BLK, BLK), :] = blk
        s = jnp.sum(blk, axis=0, keepdims=True)
        q = jnp.sum(blk * blk, axis=0, keepdims=True)

        @pl.when(i == 0)
        def _():
            acc_s[...] = s
            acc_q[...] = q

        @pl.when(i > 0)
        def _():
            acc_s[...] += s
            acc_q[...] += q

    @pl.when(p == 1)
    def _():
        m = acc_s[...] / N
        v = acc_q[...] / N - m * m
        h = (x_sc[pl.ds(i * BLK, BLK), :] - m) * (lax.rsqrt(v + EPS) * g_ref[...]) + b_ref[...]
        h = h * dv_sc[pl.ds(i * BLK, BLK), :]
        xw_ref[...] = jnp.dot(h, w_ref[...], preferred_element_type=jnp.float32,
                              precision=lax.Precision.HIGHEST)


def _k_prep(x, degp, g2, b2, W):
    return pl.pallas_call(
        _prep_body,
        grid=(2, GRID),
        in_specs=[
            pl.BlockSpec((BLK, D), lambda p, i: (i * (1 - p), 0)),
            pl.BlockSpec((NC, BLK, D), lambda p, i: (0, i * (1 - p), 0)),
            pl.BlockSpec((1, D), lambda p, i: (0, 0)),
            pl.BlockSpec((1, D), lambda p, i: (0, 0)),
            pl.BlockSpec((D, D), lambda p, i: (0, 0)),
        ],
        out_specs=[
            pl.BlockSpec((BLK, D), lambda p, i: (i * p, 0)),
            pl.BlockSpec((BLK, 1), lambda p, i: (i * (1 - p) + (GRID - 1) * p, 0)),
        ],
        out_shape=[
            jax.ShapeDtypeStruct((N, D), jnp.float32),
            jax.ShapeDtypeStruct((N, 1), jnp.float32),
        ],
        scratch_shapes=[pltpu.VMEM((N, D), jnp.float32),
                        pltpu.VMEM((N, 1), jnp.float32),
                        pltpu.VMEM((1, D), jnp.float32),
                        pltpu.VMEM((1, D), jnp.float32)],
    )(x, degp, g2, b2, W)


def _layer_body(sp_ref, xw_ref, dinv_ref, bc_ref, g_ref, b_ref, w_ref,
                out_ref, y_sc, acc_s, acc_q):
    p = pl.program_id(0)
    i = pl.program_id(1)

    @pl.when(p == 0)
    def _():
        y = sp_ref[0] + sp_ref[1] + xw_ref[...]
        y = jnp.maximum(y * dinv_ref[...] + bc_ref[...], 0.0)
        y_sc[pl.ds(i * BLK, BLK), :] = y
        s = jnp.sum(y, axis=0, keepdims=True)
        q = jnp.sum(y * y, axis=0, keepdims=True)

        @pl.when(i == 0)
        def _():
            acc_s[...] = s
            acc_q[...] = q

        @pl.when(i > 0)
        def _():
            acc_s[...] += s
            acc_q[...] += q

    @pl.when(p == 1)
    def _():
        m = acc_s[...] / N
        v = acc_q[...] / N - m * m
        h = (y_sc[pl.ds(i * BLK, BLK), :] - m) * (lax.rsqrt(v + EPS) * g_ref[...]) + b_ref[...]
        h = h * dinv_ref[...]
        out_ref[...] = jnp.dot(h, w_ref[...], preferred_element_type=jnp.float32,
                               precision=lax.Precision.HIGHEST)


def _k_layer(sp, xw, dinv, bc2, g2, b2, W):
    return pl.pallas_call(
        _layer_body,
        grid=(2, GRID),
        in_specs=[
            pl.BlockSpec((NC, BLK, D), lambda p, i: (0, i * (1 - p), 0)),
            pl.BlockSpec((BLK, D), lambda p, i: (i * (1 - p), 0)),
            pl.BlockSpec((BLK, 1), lambda p, i: (i, 0)),
            pl.BlockSpec((1, D), lambda p, i: (0, 0)),
            pl.BlockSpec((1, D), lambda p, i: (0, 0)),
            pl.BlockSpec((1, D), lambda p, i: (0, 0)),
            pl.BlockSpec((D, D), lambda p, i: (0, 0)),
        ],
        out_specs=pl.BlockSpec((BLK, D), lambda p, i: (i * p, 0)),
        out_shape=jax.ShapeDtypeStruct((N, D), jnp.float32),
        scratch_shapes=[pltpu.VMEM((N, D), jnp.float32),
                        pltpu.VMEM((1, D), jnp.float32),
                        pltpu.VMEM((1, D), jnp.float32)],
    )(sp, xw, dinv, bc2, g2, b2, W)


def _final_body(sp_ref, xw_ref, dinv_ref, b_ref, w_ref, ob_ref, out_ref):
    y = sp_ref[0] + sp_ref[1] + xw_ref[...]
    y = jnp.maximum(y * dinv_ref[...] + b_ref[...], 0.0)
    out_ref[...] = jnp.dot(y, w_ref[...], preferred_element_type=jnp.float32,
                           precision=lax.Precision.HIGHEST) + ob_ref[...]


def _k_final(sp, xw, dinv, b2, wp, obp):
    return pl.pallas_call(
        _final_body,
        grid=(GRID,),
        in_specs=[
            pl.BlockSpec((NC, BLK, D), lambda i: (0, i, 0)),
            pl.BlockSpec((BLK, D), lambda i: (i, 0)),
            pl.BlockSpec((BLK, 1), lambda i: (i, 0)),
            pl.BlockSpec((1, D), lambda i: (0, 0)),
            pl.BlockSpec((D, NCLS), lambda i: (0, 0)),
            pl.BlockSpec((1, NCLS), lambda i: (0, 0)),
        ],
        out_specs=pl.BlockSpec((BLK, NCLS), lambda i: (i, 0)),
        out_shape=jax.ShapeDtypeStruct((N, NCLS), jnp.float32),
    )(sp, xw, dinv, b2, wp, obp)


def kernel(x, edge_index, bn1_g, bn1_b, W1, b1, bn2_g, bn2_b, W2, b2,
           bn3_g, bn3_b, W3, b3, out_W, out_b):
    f32 = jnp.float32
    src2 = edge_index[0].reshape(NW, EPW)
    dst3 = edge_index[1].reshape(NW, NB, B)
    zeros_nd = jnp.zeros((NP, D), f32)
    ones_bd = jnp.ones((B, D), f32)
    wp = out_W
    obp = out_b.reshape(1, NCLS)
    r2 = lambda a: a.reshape(1, D)

    degp = _sc_deg(dst3, zeros_nd, ones_bd)
    xw1, dinv = _k_prep(x, degp, r2(bn1_g), r2(bn1_b), W1)
    sp1 = _sc_spmm(xw1, src2, dst3, zeros_nd)
    xw2 = _k_layer(sp1, xw1, dinv, r2(b1), r2(bn2_g), r2(bn2_b), W2)
    sp2 = _sc_spmm(xw2, src2, dst3, zeros_nd)
    xw3 = _k_layer(sp2, xw2, dinv, r2(b2), r2(bn3_g), r2(bn3_b), W3)
    sp3 = _sc_spmm(xw3, src2, dst3, zeros_nd)
    return _k_final(sp3, xw3, dinv, r2(b3), wp, obp)
